# baseline (speedup 1.0000x reference)
import functools

import jax
import jax.numpy as jnp
from jax.experimental import pallas as pl
from jax.experimental.pallas import tpu as pltpu
from jax.experimental.pallas import tpu_sc as plsc

_LANES = 16
_WINDOW = 128


def _sc_gather_rows(table, idx):
    n, d = idx.shape[0], table.shape[1]
    nwin = n // _WINDOW
    mesh = plsc.VectorSubcoreMesh(core_axis_name="c", subcore_axis_name="s")

    @functools.partial(
        pl.kernel,
        out_type=jax.ShapeDtypeStruct((n, d), table.dtype),
        mesh=mesh,
        compiler_params=pltpu.CompilerParams(use_tc_tiling_on_sc=False),
    )
    def gather_kernel(table_hbm, idx_hbm, out_hbm):
        def body(idx_vmem, out_vmem):
            pltpu.sync_copy(table_hbm.at[idx_vmem.at[0]], out_vmem)

        pltpu.emit_pipeline(
            body,
            grid=(nwin,),
            in_specs=[pl.BlockSpec((1, _WINDOW), lambda i: (0, i))],
            out_specs=[pl.BlockSpec((_WINDOW, d), lambda i: (i, 0))],
            core_axis_name=("c", "s"),
            dimension_semantics=(pltpu.PARALLEL,),
        )(idx_hbm, out_hbm)

    return gather_kernel(table, idx.reshape(1, n))


def _dist_body(g_ref, w_ref, o0_ref, o1_ref, o2_ref):
    gt = jnp.transpose(g_ref[...], (1, 0))
    bt = jnp.transpose(w_ref[...], (1, 0))
    outs = (o0_ref, o1_ref, o2_ref)
    for k in range(3):
        p = []
        for c in range(3):
            acc = bt[3 * k + 0] * gt[16 * k + 0 + c]
            acc = acc + bt[3 * k + 1] * gt[16 * k + 3 + c]
            acc = acc + bt[3 * k + 2] * gt[16 * k + 6 + c]
            p.append(acc)
        d2 = p[0] * p[0] + p[1] * p[1] + p[2] * p[2]
        outs[k][...] = jnp.sqrt(d2).reshape(1, 1, -1)


def _dist(g, bary_flat, npix):
    blk = 2048
    grid = npix // blk
    out_sds = jax.ShapeDtypeStruct((grid, 1, blk), jnp.float32)
    outs = pl.pallas_call(
        _dist_body,
        grid=(grid,),
        in_specs=[
            pl.BlockSpec((blk, 48), lambda i: (i, 0)),
            pl.BlockSpec((blk, 9), lambda i: (i, 0)),
        ],
        out_specs=[pl.BlockSpec((1, 1, blk), lambda i: (i, 0, 0))] * 3,
        out_shape=[out_sds] * 3,
    )(g.reshape(npix, 48), bary_flat.reshape(npix, 9))
    return [o.reshape(npix) for o in outs]


def kernel(pix_to_face, bary_coords, verts, faces):
    n, h, w, k = pix_to_face.shape
    f = faces.shape[0]
    b = n * h * w * k

    verts_pad = jnp.pad(verts.astype(jnp.float32), ((0, 0), (0, _LANES - 3)))
    faces32 = faces.astype(jnp.int32)
    fp = ((f + _WINDOW - 1) // _WINDOW) * _WINDOW
    faces_pad = jnp.pad(faces32, ((0, fp - f), (0, 0)))
    corner_idx = faces_pad.T.reshape(-1)
    corner_rows = _sc_gather_rows(verts_pad, corner_idx)
    tbl = jnp.concatenate(
        [corner_rows[0 * fp:0 * fp + f, 0:3],
         corner_rows[1 * fp:1 * fp + f, 0:3],
         corner_rows[2 * fp:2 * fp + f, 0:3],
         jnp.zeros((f, _LANES - 9), jnp.float32)], axis=1)

    idx = pix_to_face.astype(jnp.int32).reshape(-1)
    g = _sc_gather_rows(tbl, idx)

    npix = n * h * w
    d = _dist(g, bary_coords.astype(jnp.float32), npix)
    return tuple(d[i].reshape(n, h, w, 1) for i in range(k))
